# baseline (speedup 1.0000x reference)
import dataclasses
import functools

import jax
import jax.numpy as jnp
from jax import lax
from jax.experimental import pallas as pl
from jax.experimental.pallas import tpu as pltpu
from jax.experimental.pallas import tpu_sc as plsc

NUM_USER = 25000
NUM_ITEM = 25000
N_NODES = NUM_USER + NUM_ITEM
NUM_LAYER = 3
D = 64
E_EDGES = 800000

NUM_SC = 2
NUM_SUBCORES = 16
HALF = N_NODES // NUM_SC
ACC_ROWS = 25088
ROWS_PER_TILE = ACC_ROWS // NUM_SUBCORES
CHUNK = 400
NUM_TILES = NUM_SC * NUM_SUBCORES
N_QCHUNKS = E_EDGES // CHUNK
CAP = 25600
STG = 800

_BLK = 1000
_GRID = NUM_ITEM // _BLK


def _row_normalize(v):
    n = jnp.sqrt(jnp.sum(v * v, axis=1, keepdims=True))
    return v / jnp.maximum(n, 1e-12)


def _prologue_body(f_ref, id_ref, w1_ref, b1_ref, w2_ref, b2_ref, p_ref, o_ref):
    f = f_ref[...]
    t = jnp.dot(f, w1_ref[...], preferred_element_type=jnp.float32) + b1_ref[...]
    t = jnp.where(t >= 0, t, 0.01 * t)
    t = jnp.dot(t, w2_ref[...], preferred_element_type=jnp.float32) + b2_ref[...]
    idb = id_ref[...]
    t = jnp.sqrt(jnp.abs((idb * idb + t * t) * 0.5 + 1e-8))
    o_ref[0] = _row_normalize(p_ref[...])
    o_ref[1] = _row_normalize(t)


def _dense_prologue(features, id_embd, W1, b1, W2, b2, preference):
    x2 = pl.pallas_call(
        _prologue_body,
        grid=(_GRID,),
        in_specs=[
            pl.BlockSpec((_BLK, 128), lambda i: (i, 0)),
            pl.BlockSpec((_BLK, D), lambda i: (i, 0)),
            pl.BlockSpec((128, 256), lambda i: (0, 0)),
            pl.BlockSpec((1, 256), lambda i: (0, 0)),
            pl.BlockSpec((256, D), lambda i: (0, 0)),
            pl.BlockSpec((1, D), lambda i: (0, 0)),
            pl.BlockSpec((_BLK, D), lambda i: (i, 0)),
        ],
        out_specs=pl.BlockSpec((2, _BLK, D), lambda i: (0, i, 0)),
        out_shape=jax.ShapeDtypeStruct((2, NUM_ITEM, D), jnp.float32),
    )(features, id_embd, W1, b1.reshape(1, -1), W2, b2.reshape(1, -1), preference)
    return x2.reshape(N_NODES, D)


def _sc_compiler_params():
    cp = pltpu.CompilerParams()
    if "needs_layout_passes" in pltpu.CompilerParams.__dataclass_fields__:
        cp = dataclasses.replace(cp, needs_layout_passes=False)
    if "use_tc_tiling_on_sc" in pltpu.CompilerParams.__dataclass_fields__:
        cp = dataclasses.replace(cp, use_tc_tiling_on_sc=False)
    return cp


def _partition_body(src_hbm, dst_hbm, val_hbm,
                    srcp_hbm, dstp_hbm, valp_hbm, cnts_hbm,
                    src_v, dst_v, val_v,
                    stg_s0, stg_d0, stg_v0, stg_s1, stg_d1, stg_v1,
                    cnt_v, sem_s, sem_d, sem_v):
    c = lax.axis_index("c")
    s = lax.axis_index("s")
    w = c * NUM_SUBCORES + s
    nch = jnp.where(w < N_QCHUNKS - 62 * NUM_TILES, 63, 62)

    stgs = (stg_s0, stg_s1)
    stgd = (stg_d0, stg_d1)
    stgv = (stg_v0, stg_v1)

    def flush(h, off):
        off = pl.multiple_of(off, CHUNK)
        pltpu.sync_copy(stgs[h].at[pl.ds(0, CHUNK)],
                        srcp_hbm.at[h, w, pl.ds(off, CHUNK)])
        pltpu.sync_copy(stgd[h].at[pl.ds(0, CHUNK)],
                        dstp_hbm.at[h, w, pl.ds(off, CHUNK)])
        pltpu.sync_copy(stgv[h].at[pl.ds(0, CHUNK)],
                        valp_hbm.at[h, w, pl.ds(off, CHUNK)])

    def shift(h):
        @pl.loop(0, CHUNK // 16)
        def _sh(i):
            sl_from = pl.ds(CHUNK + i * 16, 16)
            sl_to = pl.ds(i * 16, 16)
            stgs[h][sl_to] = stgs[h][sl_from]
            stgd[h][sl_to] = stgd[h][sl_from]
            stgv[h][sl_to] = stgv[h][sl_from]

    def chunk_body(j, carry):
        c0, c1, o0, o1 = carry
        off_in = (w + j * NUM_TILES) * CHUNK
        cp_s = pltpu.async_copy(src_hbm.at[pl.ds(off_in, CHUNK)], src_v, sem_s)
        cp_d = pltpu.async_copy(dst_hbm.at[pl.ds(off_in, CHUNK)], dst_v, sem_d)
        cp_v = pltpu.async_copy(val_hbm.at[pl.ds(off_in, CHUNK)], val_v, sem_v)
        cp_s.wait()
        cp_d.wait()
        cp_v.wait()

        def slice_body(i, cc):
            c0i, c1i = cc
            sl = pl.ds(i * 16, 16)
            dd = dst_v[sl]
            ss = src_v[sl]
            vv = val_v[sl]
            m0 = dd < HALF
            m1 = jnp.logical_not(m0)
            plsc.store_compressed(stg_s0.at[pl.ds(c0i, 16)], ss, mask=m0)
            plsc.store_compressed(stg_d0.at[pl.ds(c0i, 16)], dd, mask=m0)
            plsc.store_compressed(stg_v0.at[pl.ds(c0i, 16)], vv, mask=m0)
            plsc.store_compressed(stg_s1.at[pl.ds(c1i, 16)], ss, mask=m1)
            plsc.store_compressed(stg_d1.at[pl.ds(c1i, 16)], dd - HALF, mask=m1)
            plsc.store_compressed(stg_v1.at[pl.ds(c1i, 16)], vv, mask=m1)
            n0 = jnp.sum(m0.astype(jnp.int32))
            return (c0i + n0, c1i + (16 - n0))

        c0, c1 = lax.fori_loop(0, CHUNK // 16, slice_body, (c0, c1))

        f0 = c0 >= CHUNK

        @pl.when(f0)
        def _():
            flush(0, o0)
            shift(0)

        o0 = o0 + jnp.where(f0, CHUNK, 0)
        c0 = c0 - jnp.where(f0, CHUNK, 0)

        f1 = c1 >= CHUNK

        @pl.when(f1)
        def _():
            flush(1, o1)
            shift(1)

        o1 = o1 + jnp.where(f1, CHUNK, 0)
        c1 = c1 - jnp.where(f1, CHUNK, 0)
        return (c0, c1, o0, o1)

    c0, c1, o0, o1 = lax.fori_loop(
        0, nch, chunk_body, (jnp.int32(0), jnp.int32(0), jnp.int32(0), jnp.int32(0)))

    for h in range(2):
        cc = (c0, c1)[h]
        oo = (o0, o1)[h]

        @pl.loop(0, CHUNK // 16)
        def _pad(i, _h=h, _cc=cc):
            sl = pl.ds(_cc + i * 16, 16)
            stgs[_h][sl] = jnp.zeros((16,), jnp.int32)
            stgd[_h][sl] = jnp.full((16,), HALF, jnp.int32)
            stgv[_h][sl] = jnp.zeros((16,), jnp.float32)

        flush(h, oo)
        cnt_v[pl.ds(0, 16)] = jnp.full((16,), oo + CHUNK, jnp.int32)
        pltpu.sync_copy(cnt_v.at[pl.ds(0, 8)], cnts_hbm.at[h, w])


def _partition_sc(src, dst, val):
    mesh = plsc.VectorSubcoreMesh(core_axis_name="c", subcore_axis_name="s")
    k = pl.kernel(
        _partition_body,
        out_type=[
            jax.ShapeDtypeStruct((2, NUM_TILES, CAP), jnp.int32),
            jax.ShapeDtypeStruct((2, NUM_TILES, CAP), jnp.int32),
            jax.ShapeDtypeStruct((2, NUM_TILES, CAP), jnp.float32),
            jax.ShapeDtypeStruct((2, NUM_TILES, 8), jnp.int32),
        ],
        mesh=mesh,
        scratch_types=[
            pltpu.VMEM((CHUNK,), jnp.int32),
            pltpu.VMEM((CHUNK,), jnp.int32),
            pltpu.VMEM((CHUNK,), jnp.float32),
            pltpu.VMEM((STG,), jnp.int32),
            pltpu.VMEM((STG,), jnp.int32),
            pltpu.VMEM((STG,), jnp.float32),
            pltpu.VMEM((STG,), jnp.int32),
            pltpu.VMEM((STG,), jnp.int32),
            pltpu.VMEM((STG,), jnp.float32),
            pltpu.VMEM((16,), jnp.int32),
            pltpu.SemaphoreType.DMA,
            pltpu.SemaphoreType.DMA,
            pltpu.SemaphoreType.DMA,
        ],
        compiler_params=_sc_compiler_params(),
    )
    return k(src, dst, val)


def _spmm_body(x_hbm, srcp_hbm, dstp_hbm, valp_hbm, cnts_hbm, y_hbm,
               src_v, dst_v, val_v, rows_v, cnt_v, acc_sh,
               sem_s, sem_d, sem_v, sem_g):
    c = lax.axis_index("c")
    s = lax.axis_index("s")

    @pl.loop(0, CHUNK)
    def _zero_rows(i):
        for k in range(D // 16):
            rows_v[i, pl.ds(k * 16, 16)] = jnp.zeros((16,), jnp.float32)

    for r in range(3):
        pltpu.sync_copy(rows_v, acc_sh.at[pl.ds(s * ROWS_PER_TILE + r * CHUNK, CHUNK)])
    pltpu.sync_copy(rows_v.at[pl.ds(0, ROWS_PER_TILE - 3 * CHUNK)],
                    acc_sh.at[pl.ds(s * ROWS_PER_TILE + 3 * CHUNK,
                                    ROWS_PER_TILE - 3 * CHUNK)])
    plsc.subcore_barrier()

    for rr in range(2):
        region = s * 2 + rr
        pltpu.sync_copy(cnts_hbm.at[c, region], cnt_v.at[pl.ds(0, 8)])
        n = cnt_v[pl.ds(0, 16)][0] // CHUNK

        def chunk_body(j, _):
            off = j * CHUNK
            cp_s = pltpu.async_copy(srcp_hbm.at[c, region, pl.ds(off, CHUNK)],
                                    src_v, sem_s)
            cp_d = pltpu.async_copy(dstp_hbm.at[c, region, pl.ds(off, CHUNK)],
                                    dst_v, sem_d)
            cp_v = pltpu.async_copy(valp_hbm.at[c, region, pl.ds(off, CHUNK)],
                                    val_v, sem_v)
            cp_s.wait()
            cp_g = pltpu.async_copy(x_hbm.at[src_v], rows_v, sem_g)
            cp_d.wait()
            cp_v.wait()
            cp_g.wait()

            @plsc.parallel_loop(0, CHUNK, unroll=8)
            def _scale(i):
                vv = plsc.load_gather(val_v, [jnp.full((16,), i, jnp.int32)])
                for k in range(D // 16):
                    sl = pl.ds(k * 16, 16)
                    rows_v[i, sl] = rows_v[i, sl] * vv

            pltpu.sync_copy(rows_v, acc_sh.at[dst_v], add=True)
            return 0

        lax.fori_loop(0, n, chunk_body, 0)

    plsc.subcore_barrier()

    pltpu.sync_copy(acc_sh.at[pl.ds(s * ROWS_PER_TILE, ROWS_PER_TILE)],
                    y_hbm.at[c, pl.ds(s * ROWS_PER_TILE, ROWS_PER_TILE)])


def _spmm_sc(x, srcp, dstp, valp, cnts):
    mesh = plsc.VectorSubcoreMesh(core_axis_name="c", subcore_axis_name="s")
    k = pl.kernel(
        _spmm_body,
        out_type=jax.ShapeDtypeStruct((NUM_SC, ACC_ROWS, D), jnp.float32),
        mesh=mesh,
        scratch_types=[
            pltpu.VMEM((CHUNK,), jnp.int32),
            pltpu.VMEM((CHUNK,), jnp.int32),
            pltpu.VMEM((CHUNK,), jnp.float32),
            pltpu.VMEM((CHUNK, D), jnp.float32),
            pltpu.VMEM((16,), jnp.int32),
            pltpu.VMEM_SHARED((ACC_ROWS, D), jnp.float32),
            pltpu.SemaphoreType.DMA,
            pltpu.SemaphoreType.DMA,
            pltpu.SemaphoreType.DMA,
            pltpu.SemaphoreType.DMA,
        ],
        compiler_params=_sc_compiler_params(),
    )
    return k(x, srcp, dstp, valp, cnts)


def _rescale_body(y_ref, ego_ref, acc_ref, x_ref, accout_ref):
    y = y_ref[0]
    e = ego_ref[...]
    num = jnp.sum(y * e, axis=1, keepdims=True)
    n1 = jnp.maximum(jnp.sqrt(jnp.sum(y * y, axis=1, keepdims=True)), 1e-8)
    n2 = jnp.maximum(jnp.sqrt(jnp.sum(e * e, axis=1, keepdims=True)), 1e-8)
    w = num / (n1 * n2)
    xn = w * y
    x_ref[...] = xn
    accout_ref[...] = acc_ref[...] + xn


def _rescale(y2, ego, acc):
    return pl.pallas_call(
        _rescale_body,
        grid=(2, HALF // _BLK),
        in_specs=[
            pl.BlockSpec((1, _BLK, D), lambda c, j: (c, j, 0)),
            pl.BlockSpec((_BLK, D), lambda c, j: (c * (HALF // _BLK) + j, 0)),
            pl.BlockSpec((_BLK, D), lambda c, j: (c * (HALF // _BLK) + j, 0)),
        ],
        out_specs=[
            pl.BlockSpec((_BLK, D), lambda c, j: (c * (HALF // _BLK) + j, 0)),
            pl.BlockSpec((_BLK, D), lambda c, j: (c * (HALF // _BLK) + j, 0)),
        ],
        out_shape=[
            jax.ShapeDtypeStruct((N_NODES, D), jnp.float32),
            jax.ShapeDtypeStruct((N_NODES, D), jnp.float32),
        ],
    )(y2, ego, acc)


def kernel(features, id_embd, adj_indices, adj_values, W1, b1, W2, b2, preference):
    x0 = _dense_prologue(features, id_embd, W1, b1, W2, b2, preference)
    dst = adj_indices[0].astype(jnp.int32)
    src = adj_indices[1].astype(jnp.int32)
    val = adj_values
    srcp, dstp, valp, cnts = _partition_sc(src, dst, val)
    x = x0
    acc = x0
    for _ in range(NUM_LAYER):
        y2 = _spmm_sc(x, srcp, dstp, valp, cnts)
        x, acc = _rescale(y2, x0, acc)
    return (acc, preference)


# speedup vs baseline: 1.0178x; 1.0178x over previous
import dataclasses
import functools

import jax
import jax.numpy as jnp
from jax import lax
from jax.experimental import pallas as pl
from jax.experimental.pallas import tpu as pltpu
from jax.experimental.pallas import tpu_sc as plsc

NUM_USER = 25000
NUM_ITEM = 25000
N_NODES = NUM_USER + NUM_ITEM
NUM_LAYER = 3
D = 64
E_EDGES = 800000

NUM_SC = 2
NUM_SUBCORES = 16
HALF = N_NODES // NUM_SC
ACC_ROWS = 25088
ROWS_PER_TILE = ACC_ROWS // NUM_SUBCORES
CHUNK = 400
NUM_TILES = NUM_SC * NUM_SUBCORES
N_QCHUNKS = E_EDGES // CHUNK
SCH = 96
SUP = 4 * SCH
BLK_E = 12 * SCH
FL = BLK_E
CAP = 23 * FL
STG = 2 * FL

_BLK = 1000
_GRID = NUM_ITEM // _BLK


def _row_normalize(v):
    n = jnp.sqrt(jnp.sum(v * v, axis=1, keepdims=True))
    return v / jnp.maximum(n, 1e-12)


def _prologue_body(f_ref, id_ref, w1_ref, b1_ref, w2_ref, b2_ref, p_ref, o_ref):
    f = f_ref[...]
    t = jnp.dot(f, w1_ref[...], preferred_element_type=jnp.float32) + b1_ref[...]
    t = jnp.where(t >= 0, t, 0.01 * t)
    t = jnp.dot(t, w2_ref[...], preferred_element_type=jnp.float32) + b2_ref[...]
    idb = id_ref[...]
    t = jnp.sqrt(jnp.abs((idb * idb + t * t) * 0.5 + 1e-8))
    o_ref[0] = _row_normalize(p_ref[...])
    o_ref[1] = _row_normalize(t)


def _dense_prologue(features, id_embd, W1, b1, W2, b2, preference):
    x2 = pl.pallas_call(
        _prologue_body,
        grid=(_GRID,),
        in_specs=[
            pl.BlockSpec((_BLK, 128), lambda i: (i, 0)),
            pl.BlockSpec((_BLK, D), lambda i: (i, 0)),
            pl.BlockSpec((128, 256), lambda i: (0, 0)),
            pl.BlockSpec((1, 256), lambda i: (0, 0)),
            pl.BlockSpec((256, D), lambda i: (0, 0)),
            pl.BlockSpec((1, D), lambda i: (0, 0)),
            pl.BlockSpec((_BLK, D), lambda i: (i, 0)),
        ],
        out_specs=pl.BlockSpec((2, _BLK, D), lambda i: (0, i, 0)),
        out_shape=jax.ShapeDtypeStruct((2, NUM_ITEM, D), jnp.float32),
    )(features, id_embd, W1, b1.reshape(1, -1), W2, b2.reshape(1, -1), preference)
    return x2.reshape(N_NODES, D)


def _sc_compiler_params():
    cp = pltpu.CompilerParams()
    if "needs_layout_passes" in pltpu.CompilerParams.__dataclass_fields__:
        cp = dataclasses.replace(cp, needs_layout_passes=False)
    if "use_tc_tiling_on_sc" in pltpu.CompilerParams.__dataclass_fields__:
        cp = dataclasses.replace(cp, use_tc_tiling_on_sc=False)
    return cp


def _partition_body(src_hbm, dst_hbm, val_hbm,
                    srcp_hbm, dstp_hbm, valp_hbm, cnts_hbm,
                    src_v, dst_v, val_v,
                    stg_s0, stg_d0, stg_v0, stg_s1, stg_d1, stg_v1,
                    cnt_v, sem_s, sem_d, sem_v):
    c = lax.axis_index("c")
    s = lax.axis_index("s")
    w = c * NUM_SUBCORES + s
    nch = jnp.where(w < N_QCHUNKS - 62 * NUM_TILES, 63, 62)

    stgs = (stg_s0, stg_s1)
    stgd = (stg_d0, stg_d1)
    stgv = (stg_v0, stg_v1)

    def flush(h, off):
        off = pl.multiple_of(off, FL)
        pltpu.sync_copy(stgs[h].at[pl.ds(0, FL)],
                        srcp_hbm.at[h, w, pl.ds(off, FL)])
        pltpu.sync_copy(stgd[h].at[pl.ds(0, FL)],
                        dstp_hbm.at[h, w, pl.ds(off, FL)])
        pltpu.sync_copy(stgv[h].at[pl.ds(0, FL)],
                        valp_hbm.at[h, w, pl.ds(off, FL)])

    def shift(h):
        @pl.loop(0, FL // 16)
        def _sh(i):
            sl_from = pl.ds(FL + i * 16, 16)
            sl_to = pl.ds(i * 16, 16)
            stgs[h][sl_to] = stgs[h][sl_from]
            stgd[h][sl_to] = stgd[h][sl_from]
            stgv[h][sl_to] = stgv[h][sl_from]

    def chunk_body(j, carry):
        c0, c1, o0, o1 = carry
        off_in = (w + j * NUM_TILES) * CHUNK
        cp_s = pltpu.async_copy(src_hbm.at[pl.ds(off_in, CHUNK)], src_v, sem_s)
        cp_d = pltpu.async_copy(dst_hbm.at[pl.ds(off_in, CHUNK)], dst_v, sem_d)
        cp_v = pltpu.async_copy(val_hbm.at[pl.ds(off_in, CHUNK)], val_v, sem_v)
        cp_s.wait()
        cp_d.wait()
        cp_v.wait()

        def slice_body(i, cc):
            c0i, c1i = cc
            sl = pl.ds(i * 16, 16)
            dd = dst_v[sl]
            ss = src_v[sl]
            vv = val_v[sl]
            m0 = dd < HALF
            m1 = jnp.logical_not(m0)
            plsc.store_compressed(stg_s0.at[pl.ds(c0i, 16)], ss, mask=m0)
            plsc.store_compressed(stg_d0.at[pl.ds(c0i, 16)], dd, mask=m0)
            plsc.store_compressed(stg_v0.at[pl.ds(c0i, 16)], vv, mask=m0)
            plsc.store_compressed(stg_s1.at[pl.ds(c1i, 16)], ss, mask=m1)
            plsc.store_compressed(stg_d1.at[pl.ds(c1i, 16)], dd - HALF, mask=m1)
            plsc.store_compressed(stg_v1.at[pl.ds(c1i, 16)], vv, mask=m1)
            n0 = jnp.sum(m0.astype(jnp.int32))
            return (c0i + n0, c1i + (16 - n0))

        c0, c1 = lax.fori_loop(0, CHUNK // 16, slice_body, (c0, c1))

        f0 = c0 >= FL

        @pl.when(f0)
        def _():
            flush(0, o0)
            shift(0)

        o0 = o0 + jnp.where(f0, FL, 0)
        c0 = c0 - jnp.where(f0, FL, 0)

        f1 = c1 >= FL

        @pl.when(f1)
        def _():
            flush(1, o1)
            shift(1)

        o1 = o1 + jnp.where(f1, FL, 0)
        c1 = c1 - jnp.where(f1, FL, 0)
        return (c0, c1, o0, o1)

    c0, c1, o0, o1 = lax.fori_loop(
        0, nch, chunk_body, (jnp.int32(0), jnp.int32(0), jnp.int32(0), jnp.int32(0)))

    for h in range(2):
        cc = (c0, c1)[h]
        oo = (o0, o1)[h]

        @pl.loop(0, FL // 16)
        def _pad(i, _h=h, _cc=cc):
            sl = pl.ds(_cc + i * 16, 16)
            stgs[_h][sl] = jnp.zeros((16,), jnp.int32)
            stgd[_h][sl] = jnp.full((16,), HALF, jnp.int32)
            stgv[_h][sl] = jnp.zeros((16,), jnp.float32)

        flush(h, oo)
        cnt_v[pl.ds(0, 16)] = jnp.full((16,), oo + FL, jnp.int32)
        pltpu.sync_copy(cnt_v.at[pl.ds(0, 8)], cnts_hbm.at[h, w])


def _partition_sc(src, dst, val):
    mesh = plsc.VectorSubcoreMesh(core_axis_name="c", subcore_axis_name="s")
    k = pl.kernel(
        _partition_body,
        out_type=[
            jax.ShapeDtypeStruct((2, NUM_TILES, CAP), jnp.int32),
            jax.ShapeDtypeStruct((2, NUM_TILES, CAP), jnp.int32),
            jax.ShapeDtypeStruct((2, NUM_TILES, CAP), jnp.float32),
            jax.ShapeDtypeStruct((2, NUM_TILES, 8), jnp.int32),
        ],
        mesh=mesh,
        scratch_types=[
            pltpu.VMEM((CHUNK,), jnp.int32),
            pltpu.VMEM((CHUNK,), jnp.int32),
            pltpu.VMEM((CHUNK,), jnp.float32),
            pltpu.VMEM((STG,), jnp.int32),
            pltpu.VMEM((STG,), jnp.int32),
            pltpu.VMEM((STG,), jnp.float32),
            pltpu.VMEM((STG,), jnp.int32),
            pltpu.VMEM((STG,), jnp.int32),
            pltpu.VMEM((STG,), jnp.float32),
            pltpu.VMEM((16,), jnp.int32),
            pltpu.SemaphoreType.DMA,
            pltpu.SemaphoreType.DMA,
            pltpu.SemaphoreType.DMA,
        ],
        compiler_params=_sc_compiler_params(),
    )
    return k(src, dst, val)


def _spmm_body(x_hbm, srcp_hbm, dstp_hbm, valp_hbm, cnts_hbm, y_hbm,
               rows_0, rows_1, rows_2, rows_3,
               fbs_0, fbs_1, fbs_2, fbv_0, fbv_1, fbv_2,
               fbd_0, fbd_1, fbd_2, cnt_v, acc_sh,
               sg_0, sg_1, sg_2, sg_3, sc_0, sc_1, sc_2, sc_3,
               sf_0, sf_1, sf_2):
    c = lax.axis_index("c")
    s = lax.axis_index("s")
    rows = (rows_0, rows_1, rows_2, rows_3)
    fbs = (fbs_0, fbs_1, fbs_2)
    fbv = (fbv_0, fbv_1, fbv_2)
    fbd = (fbd_0, fbd_1, fbd_2)
    sg = (sg_0, sg_1, sg_2, sg_3)
    sc = (sc_0, sc_1, sc_2, sc_3)
    sf = (sf_0, sf_1, sf_2)

    @pl.loop(0, SCH)
    def _zero_rows(i):
        for k in range(D // 16):
            rows_0[i, pl.ds(k * 16, 16)] = jnp.zeros((16,), jnp.float32)

    for r in range(ROWS_PER_TILE // SCH):
        pltpu.sync_copy(rows_0, acc_sh.at[pl.ds(s * ROWS_PER_TILE + r * SCH, SCH)])
    _REM = ROWS_PER_TILE - (ROWS_PER_TILE // SCH) * SCH
    if _REM:
        pltpu.sync_copy(rows_0.at[pl.ds(0, _REM)],
                        acc_sh.at[pl.ds(s * ROWS_PER_TILE
                                        + (ROWS_PER_TILE // SCH) * SCH, _REM)])
    plsc.subcore_barrier()

    for rr in range(2):
        region = s * 2 + rr
        pltpu.sync_copy(cnts_hbm.at[c, region], cnt_v.at[pl.ds(0, 8)])
        cnt = cnt_v[pl.ds(0, 16)][0]
        nblk = cnt // BLK_E

        def f_issue(j, sup):
            sup = pl.multiple_of(sup, 1)
            pltpu.async_copy(srcp_hbm.at[c, region, pl.ds(sup * SUP, SUP)],
                             fbs[j], sf[j])
            pltpu.async_copy(valp_hbm.at[c, region, pl.ds(sup * SUP, SUP)],
                             fbv[j], sf[j])
            for q in range(4):
                pltpu.async_copy(
                    dstp_hbm.at[c, region, pl.ds(sup * SUP + q * SCH, SCH)],
                    fbd[j].at[q], sf[j])

        def f_wait(j, sup):
            pltpu.make_async_copy(srcp_hbm.at[c, region, pl.ds(sup * SUP, SUP)],
                                  fbs[j], sf[j]).wait()
            pltpu.make_async_copy(valp_hbm.at[c, region, pl.ds(sup * SUP, SUP)],
                                  fbv[j], sf[j]).wait()
            for q in range(4):
                pltpu.make_async_copy(
                    dstp_hbm.at[c, region, pl.ds(sup * SUP + q * SCH, SCH)],
                    fbd[j].at[q], sf[j]).wait()

        def g_issue(rb, j):
            pltpu.async_copy(x_hbm.at[fbs[j].at[pl.ds(rb * SCH, SCH)]],
                             rows[rb], sg[rb])

        def g_wait(rb, j):
            pltpu.make_async_copy(x_hbm.at[fbs[j].at[pl.ds(rb * SCH, SCH)]],
                                  rows[rb], sg[rb]).wait()

        def s_issue(rb, j):
            pltpu.async_copy(rows[rb], acc_sh.at[fbd[j].at[rb]], sc[rb],
                             add=True)

        def s_wait(rb, j):
            pltpu.make_async_copy(rows[rb], acc_sh.at[fbd[j].at[rb]],
                                  sc[rb]).wait()

        def scale(rb, j):
            rref = rows[rb]
            vref = fbv[j]

            @plsc.parallel_loop(0, SCH, unroll=8)
            def _scale(e):
                vv = plsc.load_gather(
                    vref, [jnp.full((16,), e + rb * SCH, jnp.int32)])
                for k in range(D // 16):
                    sl = pl.ds(k * 16, 16)
                    rref[e, sl] = rref[e, sl] * vv

        f_issue(0, 0)
        f_issue(1, 1)
        f_wait(0, 0)
        g_issue(0, 0)
        g_issue(1, 0)

        def block(p, _):
            for i in range(12):
                rb = i % 4
                j = i // 4
                rb2 = (i + 2) % 4
                j2 = ((i + 2) // 4) % 3
                jw = 2 if i < 2 else (i - 2) // 4
                if i == 2:
                    f_wait(1, 3 * p + 1)
                if i == 6:
                    f_wait(2, 3 * p + 2)
                if i == 10:
                    @pl.when(p + 1 < nblk)
                    def _():
                        f_wait(0, 3 * p + 3)
                g_wait(rb, j)
                scale(rb, j)
                s_issue(rb, j)
                if i < 2:
                    @pl.when(p > 0)
                    def _():
                        s_wait(rb2, jw)
                    g_issue(rb2, j2)
                elif i < 10:
                    s_wait(rb2, jw)
                    g_issue(rb2, j2)
                else:
                    @pl.when(p + 1 < nblk)
                    def _():
                        s_wait(rb2, jw)
                        g_issue(rb2, j2)
                if i == 2:
                    f_issue(2, 3 * p + 2)
                if i == 6:
                    @pl.when(p + 1 < nblk)
                    def _():
                        f_issue(0, 3 * p + 3)
                if i == 10:
                    @pl.when(p + 1 < nblk)
                    def _():
                        f_issue(1, 3 * p + 4)
            return 0

        lax.fori_loop(0, nblk, block, 0)
        s_wait(2, 2)
        s_wait(3, 2)

    plsc.subcore_barrier()

    pltpu.sync_copy(acc_sh.at[pl.ds(s * ROWS_PER_TILE, ROWS_PER_TILE)],
                    y_hbm.at[c, pl.ds(s * ROWS_PER_TILE, ROWS_PER_TILE)])


def _spmm_sc(x, srcp, dstp, valp, cnts):
    mesh = plsc.VectorSubcoreMesh(core_axis_name="c", subcore_axis_name="s")
    k = pl.kernel(
        _spmm_body,
        out_type=jax.ShapeDtypeStruct((NUM_SC, ACC_ROWS, D), jnp.float32),
        mesh=mesh,
        scratch_types=(
            [pltpu.VMEM((SCH, D), jnp.float32)] * 4
            + [pltpu.VMEM((SUP,), jnp.int32)] * 3
            + [pltpu.VMEM((SUP,), jnp.float32)] * 3
            + [pltpu.VMEM((4, SCH), jnp.int32)] * 3
            + [pltpu.VMEM((16,), jnp.int32)]
            + [pltpu.VMEM_SHARED((ACC_ROWS, D), jnp.float32)]
            + [pltpu.SemaphoreType.DMA] * 11
        ),
        compiler_params=_sc_compiler_params(),
    )
    return k(x, srcp, dstp, valp, cnts)


def _rescale_body(y_ref, ego_ref, acc_ref, x_ref, accout_ref):
    y = y_ref[0]
    e = ego_ref[...]
    num = jnp.sum(y * e, axis=1, keepdims=True)
    n1 = jnp.maximum(jnp.sqrt(jnp.sum(y * y, axis=1, keepdims=True)), 1e-8)
    n2 = jnp.maximum(jnp.sqrt(jnp.sum(e * e, axis=1, keepdims=True)), 1e-8)
    w = num / (n1 * n2)
    xn = w * y
    x_ref[...] = xn
    accout_ref[...] = acc_ref[...] + xn


def _rescale(y2, ego, acc):
    return pl.pallas_call(
        _rescale_body,
        grid=(2, HALF // _BLK),
        in_specs=[
            pl.BlockSpec((1, _BLK, D), lambda c, j: (c, j, 0)),
            pl.BlockSpec((_BLK, D), lambda c, j: (c * (HALF // _BLK) + j, 0)),
            pl.BlockSpec((_BLK, D), lambda c, j: (c * (HALF // _BLK) + j, 0)),
        ],
        out_specs=[
            pl.BlockSpec((_BLK, D), lambda c, j: (c * (HALF // _BLK) + j, 0)),
            pl.BlockSpec((_BLK, D), lambda c, j: (c * (HALF // _BLK) + j, 0)),
        ],
        out_shape=[
            jax.ShapeDtypeStruct((N_NODES, D), jnp.float32),
            jax.ShapeDtypeStruct((N_NODES, D), jnp.float32),
        ],
    )(y2, ego, acc)


def kernel(features, id_embd, adj_indices, adj_values, W1, b1, W2, b2, preference):
    x0 = _dense_prologue(features, id_embd, W1, b1, W2, b2, preference)
    dst = adj_indices[0].astype(jnp.int32)
    src = adj_indices[1].astype(jnp.int32)
    val = adj_values
    srcp, dstp, valp, cnts = _partition_sc(src, dst, val)
    x = x0
    acc = x0
    for _ in range(NUM_LAYER):
        y2 = _spmm_sc(x, srcp, dstp, valp, cnts)
        x, acc = _rescale(y2, x0, acc)
    return (acc, preference)


# speedup vs baseline: 1.4540x; 1.4286x over previous
import dataclasses
import functools

import jax
import jax.numpy as jnp
from jax import lax
from jax.experimental import pallas as pl
from jax.experimental.pallas import tpu as pltpu
from jax.experimental.pallas import tpu_sc as plsc

NUM_USER = 25000
NUM_ITEM = 25000
N_NODES = NUM_USER + NUM_ITEM
NUM_LAYER = 3
D = 64
E_EDGES = 800000

NUM_SC = 2
NUM_SUBCORES = 16
HALF = N_NODES // NUM_SC
ACC_ROWS = 25088
ROWS_PER_TILE = ACC_ROWS // NUM_SUBCORES
CHUNK = 400
NUM_TILES = NUM_SC * NUM_SUBCORES
N_QCHUNKS = E_EDGES // CHUNK
SCH = 96
SUP = 4 * SCH
BLK_E = 12 * SCH
FL = BLK_E
CAP = 23 * FL
STG = 2 * FL

_BLK = 1000
_GRID = NUM_ITEM // _BLK


def _row_normalize(v):
    n = jnp.sqrt(jnp.sum(v * v, axis=1, keepdims=True))
    return v / jnp.maximum(n, 1e-12)


def _prologue_body(f_ref, id_ref, w1_ref, b1_ref, w2_ref, b2_ref, p_ref, o_ref, ob_ref):
    f = f_ref[...]
    t = jnp.dot(f, w1_ref[...], preferred_element_type=jnp.float32) + b1_ref[...]
    t = jnp.where(t >= 0, t, 0.01 * t)
    t = jnp.dot(t, w2_ref[...], preferred_element_type=jnp.float32) + b2_ref[...]
    idb = id_ref[...]
    t = jnp.sqrt(jnp.abs((idb * idb + t * t) * 0.5 + 1e-8))
    xp = _row_normalize(p_ref[...])
    xt = _row_normalize(t)
    o_ref[0] = xp
    o_ref[1] = xt
    ob_ref[0] = xp.astype(jnp.bfloat16)
    ob_ref[1] = xt.astype(jnp.bfloat16)


def _dense_prologue(features, id_embd, W1, b1, W2, b2, preference):
    x2 = pl.pallas_call(
        _prologue_body,
        grid=(_GRID,),
        in_specs=[
            pl.BlockSpec((_BLK, 128), lambda i: (i, 0)),
            pl.BlockSpec((_BLK, D), lambda i: (i, 0)),
            pl.BlockSpec((128, 256), lambda i: (0, 0)),
            pl.BlockSpec((1, 256), lambda i: (0, 0)),
            pl.BlockSpec((256, D), lambda i: (0, 0)),
            pl.BlockSpec((1, D), lambda i: (0, 0)),
            pl.BlockSpec((_BLK, D), lambda i: (i, 0)),
        ],
        out_specs=[pl.BlockSpec((2, _BLK, D), lambda i: (0, i, 0)),
                   pl.BlockSpec((2, _BLK, D), lambda i: (0, i, 0))],
        out_shape=[jax.ShapeDtypeStruct((2, NUM_ITEM, D), jnp.float32),
                   jax.ShapeDtypeStruct((2, NUM_ITEM, D), jnp.bfloat16)],
    )(features, id_embd, W1, b1.reshape(1, -1), W2, b2.reshape(1, -1), preference)
    x2, xb2 = x2
    return x2.reshape(N_NODES, D), xb2.reshape(N_NODES, D)


def _sc_compiler_params():
    cp = pltpu.CompilerParams()
    if "needs_layout_passes" in pltpu.CompilerParams.__dataclass_fields__:
        cp = dataclasses.replace(cp, needs_layout_passes=False)
    if "use_tc_tiling_on_sc" in pltpu.CompilerParams.__dataclass_fields__:
        cp = dataclasses.replace(cp, use_tc_tiling_on_sc=False)
    return cp


def _partition_body(src_hbm, dst_hbm, val_hbm,
                    srcp_hbm, dstp_hbm, valp_hbm, cnts_hbm,
                    src_v, dst_v, val_v,
                    stg_s0, stg_d0, stg_v0, stg_s1, stg_d1, stg_v1,
                    cnt_v, sem_s, sem_d, sem_v):
    c = lax.axis_index("c")
    s = lax.axis_index("s")
    w = c * NUM_SUBCORES + s
    nch = jnp.where(w < N_QCHUNKS - 62 * NUM_TILES, 63, 62)

    stgs = (stg_s0, stg_s1)
    stgd = (stg_d0, stg_d1)
    stgv = (stg_v0, stg_v1)

    def flush(h, off):
        off = pl.multiple_of(off, FL)
        pltpu.sync_copy(stgs[h].at[pl.ds(0, FL)],
                        srcp_hbm.at[h, w, pl.ds(off, FL)])
        pltpu.sync_copy(stgd[h].at[pl.ds(0, FL)],
                        dstp_hbm.at[h, w, pl.ds(off, FL)])
        pltpu.sync_copy(stgv[h].at[pl.ds(0, FL)],
                        valp_hbm.at[h, w, pl.ds(off, FL)])

    def shift(h):
        @pl.loop(0, FL // 16)
        def _sh(i):
            sl_from = pl.ds(FL + i * 16, 16)
            sl_to = pl.ds(i * 16, 16)
            stgs[h][sl_to] = stgs[h][sl_from]
            stgd[h][sl_to] = stgd[h][sl_from]
            stgv[h][sl_to] = stgv[h][sl_from]

    def chunk_body(j, carry):
        c0, c1, o0, o1 = carry
        off_in = (w + j * NUM_TILES) * CHUNK
        cp_s = pltpu.async_copy(src_hbm.at[pl.ds(off_in, CHUNK)], src_v, sem_s)
        cp_d = pltpu.async_copy(dst_hbm.at[pl.ds(off_in, CHUNK)], dst_v, sem_d)
        cp_v = pltpu.async_copy(val_hbm.at[pl.ds(off_in, CHUNK)], val_v, sem_v)
        cp_s.wait()
        cp_d.wait()
        cp_v.wait()

        def slice_body(i, cc):
            c0i, c1i = cc
            sl = pl.ds(i * 16, 16)
            dd = dst_v[sl]
            ss = src_v[sl]
            vv = val_v[sl]
            m0 = dd < HALF
            m1 = jnp.logical_not(m0)
            plsc.store_compressed(stg_s0.at[pl.ds(c0i, 16)], ss, mask=m0)
            plsc.store_compressed(stg_d0.at[pl.ds(c0i, 16)], dd, mask=m0)
            plsc.store_compressed(stg_v0.at[pl.ds(c0i, 16)], vv, mask=m0)
            plsc.store_compressed(stg_s1.at[pl.ds(c1i, 16)], ss, mask=m1)
            plsc.store_compressed(stg_d1.at[pl.ds(c1i, 16)], dd - HALF, mask=m1)
            plsc.store_compressed(stg_v1.at[pl.ds(c1i, 16)], vv, mask=m1)
            n0 = jnp.sum(m0.astype(jnp.int32))
            return (c0i + n0, c1i + (16 - n0))

        c0, c1 = lax.fori_loop(0, CHUNK // 16, slice_body, (c0, c1))

        f0 = c0 >= FL

        @pl.when(f0)
        def _():
            flush(0, o0)
            shift(0)

        o0 = o0 + jnp.where(f0, FL, 0)
        c0 = c0 - jnp.where(f0, FL, 0)

        f1 = c1 >= FL

        @pl.when(f1)
        def _():
            flush(1, o1)
            shift(1)

        o1 = o1 + jnp.where(f1, FL, 0)
        c1 = c1 - jnp.where(f1, FL, 0)
        return (c0, c1, o0, o1)

    c0, c1, o0, o1 = lax.fori_loop(
        0, nch, chunk_body, (jnp.int32(0), jnp.int32(0), jnp.int32(0), jnp.int32(0)))

    for h in range(2):
        cc = (c0, c1)[h]
        oo = (o0, o1)[h]

        @pl.loop(0, FL // 16)
        def _pad(i, _h=h, _cc=cc):
            sl = pl.ds(_cc + i * 16, 16)
            stgs[_h][sl] = jnp.zeros((16,), jnp.int32)
            stgd[_h][sl] = jnp.full((16,), HALF, jnp.int32)
            stgv[_h][sl] = jnp.zeros((16,), jnp.float32)

        flush(h, oo)
        cnt_v[pl.ds(0, 16)] = jnp.full((16,), oo + FL, jnp.int32)
        pltpu.sync_copy(cnt_v.at[pl.ds(0, 8)], cnts_hbm.at[h, w])


def _partition_sc(src, dst, val):
    mesh = plsc.VectorSubcoreMesh(core_axis_name="c", subcore_axis_name="s")
    k = pl.kernel(
        _partition_body,
        out_type=[
            jax.ShapeDtypeStruct((2, NUM_TILES, CAP), jnp.int32),
            jax.ShapeDtypeStruct((2, NUM_TILES, CAP), jnp.int32),
            jax.ShapeDtypeStruct((2, NUM_TILES, CAP), jnp.float32),
            jax.ShapeDtypeStruct((2, NUM_TILES, 8), jnp.int32),
        ],
        mesh=mesh,
        scratch_types=[
            pltpu.VMEM((CHUNK,), jnp.int32),
            pltpu.VMEM((CHUNK,), jnp.int32),
            pltpu.VMEM((CHUNK,), jnp.float32),
            pltpu.VMEM((STG,), jnp.int32),
            pltpu.VMEM((STG,), jnp.int32),
            pltpu.VMEM((STG,), jnp.float32),
            pltpu.VMEM((STG,), jnp.int32),
            pltpu.VMEM((STG,), jnp.int32),
            pltpu.VMEM((STG,), jnp.float32),
            pltpu.VMEM((16,), jnp.int32),
            pltpu.SemaphoreType.DMA,
            pltpu.SemaphoreType.DMA,
            pltpu.SemaphoreType.DMA,
        ],
        compiler_params=_sc_compiler_params(),
    )
    return k(src, dst, val)


def _spmm_body(x_hbm, srcp_hbm, dstp_hbm, valp_hbm, cnts_hbm, y_hbm,
               rb_0, rb_1, rb_2, rb_3, fx_0, fx_1,
               fbs_0, fbs_1, fbs_2, fbv_0, fbv_1, fbv_2,
               fbd_0, fbd_1, fbd_2, cnt_v, acc_sh,
               sg_0, sg_1, sg_2, sg_3, ss_0, ss_1,
               sf_0, sf_1, sf_2):
    c = lax.axis_index("c")
    s = lax.axis_index("s")
    rbf = (rb_0, rb_1, rb_2, rb_3)
    fx = (fx_0, fx_1)
    fbs = (fbs_0, fbs_1, fbs_2)
    fbv = (fbv_0, fbv_1, fbv_2)
    fbd = (fbd_0, fbd_1, fbd_2)
    sg = (sg_0, sg_1, sg_2, sg_3)
    ss = (ss_0, ss_1)
    sf = (sf_0, sf_1, sf_2)

    iota16 = lax.broadcasted_iota(jnp.int32, (16,), 0)
    idx_ev = iota16 * 2
    idx_od = idx_ev + 1

    @pl.loop(0, SCH)
    def _zero_rows(i):
        for k in range(D // 16):
            fx_0[i, pl.ds(k * 16, 16)] = jnp.zeros((16,), jnp.float32)

    for r in range(ROWS_PER_TILE // SCH):
        pltpu.sync_copy(fx_0, acc_sh.at[pl.ds(s * ROWS_PER_TILE + r * SCH, SCH)])
    _REM = ROWS_PER_TILE - (ROWS_PER_TILE // SCH) * SCH
    if _REM:
        pltpu.sync_copy(fx_0.at[pl.ds(0, _REM)],
                        acc_sh.at[pl.ds(s * ROWS_PER_TILE
                                        + (ROWS_PER_TILE // SCH) * SCH, _REM)])
    plsc.subcore_barrier()

    for rr in range(2):
        region = s * 2 + rr
        pltpu.sync_copy(cnts_hbm.at[c, region], cnt_v.at[pl.ds(0, 8)])
        cnt = cnt_v[pl.ds(0, 16)][0]
        nblk = cnt // BLK_E

        def f_issue(j, sup):
            pltpu.async_copy(srcp_hbm.at[c, region, pl.ds(sup * SUP, SUP)],
                             fbs[j], sf[j])
            pltpu.async_copy(valp_hbm.at[c, region, pl.ds(sup * SUP, SUP)],
                             fbv[j], sf[j])
            for q in range(4):
                pltpu.async_copy(
                    dstp_hbm.at[c, region, pl.ds(sup * SUP + q * SCH, SCH)],
                    fbd[j].at[q], sf[j])

        def f_wait(j, sup):
            pltpu.make_async_copy(srcp_hbm.at[c, region, pl.ds(sup * SUP, SUP)],
                                  fbs[j], sf[j]).wait()
            pltpu.make_async_copy(valp_hbm.at[c, region, pl.ds(sup * SUP, SUP)],
                                  fbv[j], sf[j]).wait()
            for q in range(4):
                pltpu.make_async_copy(
                    dstp_hbm.at[c, region, pl.ds(sup * SUP + q * SCH, SCH)],
                    fbd[j].at[q], sf[j]).wait()

        def g_issue(rb, j):
            pltpu.async_copy(x_hbm.at[fbs[j].at[pl.ds(rb * SCH, SCH)]],
                             rbf[rb], sg[rb])

        def g_wait(rb, j):
            pltpu.make_async_copy(x_hbm.at[fbs[j].at[pl.ds(rb * SCH, SCH)]],
                                  rbf[rb], sg[rb]).wait()

        def s_issue(fs, j, pos):
            pltpu.async_copy(fx[fs], acc_sh.at[fbd[j].at[pos]], ss[fs],
                             add=True)

        def s_wait(fs, j, pos):
            pltpu.make_async_copy(fx[fs], acc_sh.at[fbd[j].at[pos]],
                                  ss[fs]).wait()

        def scale(rb, j, fs):
            rref = rbf[rb]
            oref = fx[fs]
            vref = fbv[j]

            @plsc.parallel_loop(0, SCH, unroll=8)
            def _scale(e):
                vv = plsc.load_gather(
                    vref, [jnp.full((16,), e + rb * SCH, jnp.int32)])
                for k in range(D // 32):
                    w = plsc.bitcast(rref[e, pl.ds(k * 32, 32)], jnp.int32)
                    ev = plsc.bitcast(w << 16, jnp.float32) * vv
                    od = plsc.bitcast(
                        w & jnp.int32(-65536), jnp.float32) * vv
                    plsc.store_scatter(oref.at[e], [idx_ev + k * 32], ev)
                    plsc.store_scatter(oref.at[e], [idx_od + k * 32], od)

        f_issue(0, 0)
        f_issue(1, 1)
        f_wait(0, 0)
        g_issue(0, 0)
        g_issue(1, 0)

        def block(p, _):
            for i in range(12):
                rb = i % 4
                j = i // 4
                rb2 = (i + 2) % 4
                j2 = ((i + 2) // 4) % 3
                fs = i % 2
                jw = 2 if i < 2 else (i - 2) // 4
                if i == 2:
                    f_wait(1, 3 * p + 1)
                if i == 6:
                    f_wait(2, 3 * p + 2)
                if i == 10:
                    @pl.when(p + 1 < nblk)
                    def _():
                        f_wait(0, 3 * p + 3)
                g_wait(rb, j)
                if i < 10:
                    g_issue(rb2, j2)
                else:
                    @pl.when(p + 1 < nblk)
                    def _():
                        g_issue(rb2, j2)
                if i < 2:
                    @pl.when(p > 0)
                    def _():
                        s_wait(fs, jw, rb2)
                else:
                    s_wait(fs, jw, rb2)
                scale(rb, j, fs)
                s_issue(fs, j, rb)
                if i == 2:
                    f_issue(2, 3 * p + 2)
                if i == 6:
                    @pl.when(p + 1 < nblk)
                    def _():
                        f_issue(0, 3 * p + 3)
                if i == 10:
                    @pl.when(p + 1 < nblk)
                    def _():
                        f_issue(1, 3 * p + 4)
            return 0

        lax.fori_loop(0, nblk, block, 0)
        s_wait(0, 2, 2)
        s_wait(1, 2, 3)

    plsc.subcore_barrier()

    pltpu.sync_copy(acc_sh.at[pl.ds(s * ROWS_PER_TILE, ROWS_PER_TILE)],
                    y_hbm.at[c, pl.ds(s * ROWS_PER_TILE, ROWS_PER_TILE)])


def _spmm_sc(x, srcp, dstp, valp, cnts):
    mesh = plsc.VectorSubcoreMesh(core_axis_name="c", subcore_axis_name="s")
    k = pl.kernel(
        _spmm_body,
        out_type=jax.ShapeDtypeStruct((NUM_SC, ACC_ROWS, D), jnp.float32),
        mesh=mesh,
        scratch_types=(
            [pltpu.VMEM((SCH, D), jnp.bfloat16)] * 4
            + [pltpu.VMEM((SCH, D), jnp.float32)] * 2
            + [pltpu.VMEM((SUP,), jnp.int32)] * 3
            + [pltpu.VMEM((SUP,), jnp.float32)] * 3
            + [pltpu.VMEM((4, SCH), jnp.int32)] * 3
            + [pltpu.VMEM((16,), jnp.int32)]
            + [pltpu.VMEM_SHARED((ACC_ROWS, D), jnp.float32)]
            + [pltpu.SemaphoreType.DMA] * 9
        ),
        compiler_params=_sc_compiler_params(),
    )
    return k(x, srcp, dstp, valp, cnts)


def _rescale_body(y_ref, ego_ref, acc_ref, x_ref, accout_ref):
    y = y_ref[0]
    e = ego_ref[...]
    num = jnp.sum(y * e, axis=1, keepdims=True)
    n1 = jnp.maximum(jnp.sqrt(jnp.sum(y * y, axis=1, keepdims=True)), 1e-8)
    n2 = jnp.maximum(jnp.sqrt(jnp.sum(e * e, axis=1, keepdims=True)), 1e-8)
    w = num / (n1 * n2)
    xn = w * y
    x_ref[...] = xn.astype(jnp.bfloat16)
    accout_ref[...] = acc_ref[...] + xn


def _rescale(y2, ego, acc):
    return pl.pallas_call(
        _rescale_body,
        grid=(2, HALF // _BLK),
        in_specs=[
            pl.BlockSpec((1, _BLK, D), lambda c, j: (c, j, 0)),
            pl.BlockSpec((_BLK, D), lambda c, j: (c * (HALF // _BLK) + j, 0)),
            pl.BlockSpec((_BLK, D), lambda c, j: (c * (HALF // _BLK) + j, 0)),
        ],
        out_specs=[
            pl.BlockSpec((_BLK, D), lambda c, j: (c * (HALF // _BLK) + j, 0)),
            pl.BlockSpec((_BLK, D), lambda c, j: (c * (HALF // _BLK) + j, 0)),
        ],
        out_shape=[
            jax.ShapeDtypeStruct((N_NODES, D), jnp.bfloat16),
            jax.ShapeDtypeStruct((N_NODES, D), jnp.float32),
        ],
    )(y2, ego, acc)


def kernel(features, id_embd, adj_indices, adj_values, W1, b1, W2, b2, preference):
    x0, x0b = _dense_prologue(features, id_embd, W1, b1, W2, b2, preference)
    dst = adj_indices[0].astype(jnp.int32)
    src = adj_indices[1].astype(jnp.int32)
    val = adj_values
    srcp, dstp, valp, cnts = _partition_sc(src, dst, val)
    xb = x0b
    acc = x0
    for _ in range(NUM_LAYER):
        y2 = _spmm_sc(xb, srcp, dstp, valp, cnts)
        xb, acc = _rescale(y2, x0, acc)
    return (acc, preference)


# speedup vs baseline: 1.5198x; 1.0452x over previous
import dataclasses
import functools

import jax
import jax.numpy as jnp
from jax import lax
from jax.experimental import pallas as pl
from jax.experimental.pallas import tpu as pltpu
from jax.experimental.pallas import tpu_sc as plsc

NUM_USER = 25000
NUM_ITEM = 25000
N_NODES = NUM_USER + NUM_ITEM
NUM_LAYER = 3
D = 64
E_EDGES = 800000

NUM_SC = 2
NUM_SUBCORES = 16
HALF = N_NODES // NUM_SC
ACC_ROWS = 25088
ROWS_PER_TILE = ACC_ROWS // NUM_SUBCORES
CHUNK = 400
NUM_TILES = NUM_SC * NUM_SUBCORES
N_QCHUNKS = E_EDGES // CHUNK
SCH = 96
SUP = 4 * SCH
BLK_E = 12 * SCH
FL = BLK_E
CAP = 23 * FL
STG = 2 * FL

_BLK = 1000
_GRID = NUM_ITEM // _BLK


def _row_normalize(v):
    n = jnp.sqrt(jnp.sum(v * v, axis=1, keepdims=True))
    return v / jnp.maximum(n, 1e-12)


def _prologue_body(f_ref, id_ref, w1_ref, b1_ref, w2_ref, b2_ref, p_ref, o_ref, ob_ref):
    f = f_ref[...]
    t = jnp.dot(f, w1_ref[...], preferred_element_type=jnp.float32) + b1_ref[...]
    t = jnp.where(t >= 0, t, 0.01 * t)
    t = jnp.dot(t, w2_ref[...], preferred_element_type=jnp.float32) + b2_ref[...]
    idb = id_ref[...]
    t = jnp.sqrt(jnp.abs((idb * idb + t * t) * 0.5 + 1e-8))
    xp = _row_normalize(p_ref[...])
    xt = _row_normalize(t)
    o_ref[0] = xp
    o_ref[1] = xt
    ob_ref[0] = xp.astype(jnp.bfloat16)
    ob_ref[1] = xt.astype(jnp.bfloat16)


def _dense_prologue(features, id_embd, W1, b1, W2, b2, preference):
    x2 = pl.pallas_call(
        _prologue_body,
        grid=(_GRID,),
        in_specs=[
            pl.BlockSpec((_BLK, 128), lambda i: (i, 0)),
            pl.BlockSpec((_BLK, D), lambda i: (i, 0)),
            pl.BlockSpec((128, 256), lambda i: (0, 0)),
            pl.BlockSpec((1, 256), lambda i: (0, 0)),
            pl.BlockSpec((256, D), lambda i: (0, 0)),
            pl.BlockSpec((1, D), lambda i: (0, 0)),
            pl.BlockSpec((_BLK, D), lambda i: (i, 0)),
        ],
        out_specs=[pl.BlockSpec((2, _BLK, D), lambda i: (0, i, 0)),
                   pl.BlockSpec((2, _BLK, D), lambda i: (0, i, 0))],
        out_shape=[jax.ShapeDtypeStruct((2, NUM_ITEM, D), jnp.float32),
                   jax.ShapeDtypeStruct((2, NUM_ITEM, D), jnp.bfloat16)],
    )(features, id_embd, W1, b1.reshape(1, -1), W2, b2.reshape(1, -1), preference)
    x2, xb2 = x2
    return x2.reshape(N_NODES, D), xb2.reshape(N_NODES, D)


def _sc_compiler_params():
    cp = pltpu.CompilerParams()
    if "needs_layout_passes" in pltpu.CompilerParams.__dataclass_fields__:
        cp = dataclasses.replace(cp, needs_layout_passes=False)
    if "use_tc_tiling_on_sc" in pltpu.CompilerParams.__dataclass_fields__:
        cp = dataclasses.replace(cp, use_tc_tiling_on_sc=False)
    return cp


def _partition_body(src_hbm, dst_hbm, val_hbm,
                    srcp_hbm, dstp_hbm, valp_hbm, cnts_hbm,
                    src_v, dst_v, val_v,
                    stg_s0, stg_d0, stg_v0, stg_s1, stg_d1, stg_v1,
                    cnt_v, sem_s, sem_d, sem_v):
    c = lax.axis_index("c")
    s = lax.axis_index("s")
    w = c * NUM_SUBCORES + s
    nch = jnp.where(w < N_QCHUNKS - 62 * NUM_TILES, 63, 62)

    stgs = (stg_s0, stg_s1)
    stgd = (stg_d0, stg_d1)
    stgv = (stg_v0, stg_v1)

    def flush(h, off):
        off = pl.multiple_of(off, FL)
        pltpu.sync_copy(stgs[h].at[pl.ds(0, FL)],
                        srcp_hbm.at[h, w, pl.ds(off, FL)])
        pltpu.sync_copy(stgd[h].at[pl.ds(0, FL)],
                        dstp_hbm.at[h, w, pl.ds(off, FL)])
        pltpu.sync_copy(stgv[h].at[pl.ds(0, FL)],
                        valp_hbm.at[h, w, pl.ds(off, FL)])

    def shift(h):
        @pl.loop(0, FL // 16)
        def _sh(i):
            sl_from = pl.ds(FL + i * 16, 16)
            sl_to = pl.ds(i * 16, 16)
            stgs[h][sl_to] = stgs[h][sl_from]
            stgd[h][sl_to] = stgd[h][sl_from]
            stgv[h][sl_to] = stgv[h][sl_from]

    def chunk_body(j, carry):
        c0, c1, o0, o1 = carry
        off_in = (w + j * NUM_TILES) * CHUNK
        cp_s = pltpu.async_copy(src_hbm.at[pl.ds(off_in, CHUNK)], src_v, sem_s)
        cp_d = pltpu.async_copy(dst_hbm.at[pl.ds(off_in, CHUNK)], dst_v, sem_d)
        cp_v = pltpu.async_copy(val_hbm.at[pl.ds(off_in, CHUNK)], val_v, sem_v)
        cp_s.wait()
        cp_d.wait()
        cp_v.wait()

        def slice_body(i, cc):
            c0i, c1i = cc
            sl = pl.ds(i * 16, 16)
            dd = dst_v[sl]
            ss = src_v[sl]
            vv = val_v[sl]
            m0 = dd < HALF
            m1 = jnp.logical_not(m0)
            plsc.store_compressed(stg_s0.at[pl.ds(c0i, 16)], ss, mask=m0)
            plsc.store_compressed(stg_d0.at[pl.ds(c0i, 16)], dd, mask=m0)
            plsc.store_compressed(stg_v0.at[pl.ds(c0i, 16)], vv, mask=m0)
            plsc.store_compressed(stg_s1.at[pl.ds(c1i, 16)], ss, mask=m1)
            plsc.store_compressed(stg_d1.at[pl.ds(c1i, 16)], dd - HALF, mask=m1)
            plsc.store_compressed(stg_v1.at[pl.ds(c1i, 16)], vv, mask=m1)
            n0 = jnp.sum(m0.astype(jnp.int32))
            return (c0i + n0, c1i + (16 - n0))

        c0, c1 = lax.fori_loop(0, CHUNK // 16, slice_body, (c0, c1))

        f0 = c0 >= FL

        @pl.when(f0)
        def _():
            flush(0, o0)
            shift(0)

        o0 = o0 + jnp.where(f0, FL, 0)
        c0 = c0 - jnp.where(f0, FL, 0)

        f1 = c1 >= FL

        @pl.when(f1)
        def _():
            flush(1, o1)
            shift(1)

        o1 = o1 + jnp.where(f1, FL, 0)
        c1 = c1 - jnp.where(f1, FL, 0)
        return (c0, c1, o0, o1)

    c0, c1, o0, o1 = lax.fori_loop(
        0, nch, chunk_body, (jnp.int32(0), jnp.int32(0), jnp.int32(0), jnp.int32(0)))

    for h in range(2):
        cc = (c0, c1)[h]
        oo = (o0, o1)[h]

        @pl.loop(0, FL // 16)
        def _pad(i, _h=h, _cc=cc):
            sl = pl.ds(_cc + i * 16, 16)
            stgs[_h][sl] = jnp.zeros((16,), jnp.int32)
            stgd[_h][sl] = jnp.full((16,), HALF, jnp.int32)
            stgv[_h][sl] = jnp.zeros((16,), jnp.float32)

        flush(h, oo)
        cnt_v[pl.ds(0, 16)] = jnp.full((16,), oo + FL, jnp.int32)
        pltpu.sync_copy(cnt_v.at[pl.ds(0, 8)], cnts_hbm.at[h, w])


def _partition_sc(src, dst, val):
    mesh = plsc.VectorSubcoreMesh(core_axis_name="c", subcore_axis_name="s")
    k = pl.kernel(
        _partition_body,
        out_type=[
            jax.ShapeDtypeStruct((2, NUM_TILES, CAP), jnp.int32),
            jax.ShapeDtypeStruct((2, NUM_TILES, CAP), jnp.int32),
            jax.ShapeDtypeStruct((2, NUM_TILES, CAP), jnp.float32),
            jax.ShapeDtypeStruct((2, NUM_TILES, 8), jnp.int32),
        ],
        mesh=mesh,
        scratch_types=[
            pltpu.VMEM((CHUNK,), jnp.int32),
            pltpu.VMEM((CHUNK,), jnp.int32),
            pltpu.VMEM((CHUNK,), jnp.float32),
            pltpu.VMEM((STG,), jnp.int32),
            pltpu.VMEM((STG,), jnp.int32),
            pltpu.VMEM((STG,), jnp.float32),
            pltpu.VMEM((STG,), jnp.int32),
            pltpu.VMEM((STG,), jnp.int32),
            pltpu.VMEM((STG,), jnp.float32),
            pltpu.VMEM((16,), jnp.int32),
            pltpu.SemaphoreType.DMA,
            pltpu.SemaphoreType.DMA,
            pltpu.SemaphoreType.DMA,
        ],
        compiler_params=_sc_compiler_params(),
    )
    return k(src, dst, val)


def _spmm_body(x_hbm, srcp_hbm, dstp_hbm, valp_hbm, cnts_hbm, y_hbm,
               rb_0, rb_1, rb_2, rb_3, fx_0, fx_1,
               fbs_0, fbs_1, fbs_2, fbv_0, fbv_1, fbv_2,
               fbd_0, fbd_1, fbd_2, cnt_v, acc_sh,
               sg_0, sg_1, sg_2, sg_3, ss_0, ss_1,
               sf_0, sf_1, sf_2):
    c = lax.axis_index("c")
    s = lax.axis_index("s")
    rbf = (rb_0, rb_1, rb_2, rb_3)
    fx = (fx_0, fx_1)
    fbs = (fbs_0, fbs_1, fbs_2)
    fbv = (fbv_0, fbv_1, fbv_2)
    fbd = (fbd_0, fbd_1, fbd_2)
    sg = (sg_0, sg_1, sg_2, sg_3)
    ss = (ss_0, ss_1)
    sf = (sf_0, sf_1, sf_2)

    iota16 = lax.broadcasted_iota(jnp.int32, (16,), 0)
    idx_ev = iota16 * 2
    idx_od = idx_ev + 1

    @pl.loop(0, SCH)
    def _zero_rows(i):
        for k in range(D // 16):
            fx_0[i, pl.ds(k * 16, 16)] = jnp.zeros((16,), jnp.float32)

    for r in range(ROWS_PER_TILE // SCH):
        pltpu.sync_copy(fx_0, acc_sh.at[pl.ds(s * ROWS_PER_TILE + r * SCH, SCH)])
    _REM = ROWS_PER_TILE - (ROWS_PER_TILE // SCH) * SCH
    if _REM:
        pltpu.sync_copy(fx_0.at[pl.ds(0, _REM)],
                        acc_sh.at[pl.ds(s * ROWS_PER_TILE
                                        + (ROWS_PER_TILE // SCH) * SCH, _REM)])
    plsc.subcore_barrier()

    for rr in range(2):
        region = s * 2 + rr
        pltpu.sync_copy(cnts_hbm.at[c, region], cnt_v.at[pl.ds(0, 8)])
        cnt = cnt_v[pl.ds(0, 16)][0]
        nblk = cnt // BLK_E

        def f_issue(j, sup):
            pltpu.async_copy(srcp_hbm.at[c, region, pl.ds(sup * SUP, SUP)],
                             fbs[j], sf[j])
            pltpu.async_copy(valp_hbm.at[c, region, pl.ds(sup * SUP, SUP)],
                             fbv[j], sf[j])
            for q in range(4):
                pltpu.async_copy(
                    dstp_hbm.at[c, region, pl.ds(sup * SUP + q * SCH, SCH)],
                    fbd[j].at[q], sf[j])

        def f_wait(j, sup):
            pltpu.make_async_copy(srcp_hbm.at[c, region, pl.ds(sup * SUP, SUP)],
                                  fbs[j], sf[j]).wait()
            pltpu.make_async_copy(valp_hbm.at[c, region, pl.ds(sup * SUP, SUP)],
                                  fbv[j], sf[j]).wait()
            for q in range(4):
                pltpu.make_async_copy(
                    dstp_hbm.at[c, region, pl.ds(sup * SUP + q * SCH, SCH)],
                    fbd[j].at[q], sf[j]).wait()

        def g_issue(rb, j):
            pltpu.async_copy(x_hbm.at[fbs[j].at[pl.ds(rb * SCH, SCH)]],
                             rbf[rb], sg[rb])

        def g_wait(rb, j):
            pltpu.make_async_copy(x_hbm.at[fbs[j].at[pl.ds(rb * SCH, SCH)]],
                                  rbf[rb], sg[rb]).wait()

        def s_issue(fs, j, pos):
            pltpu.async_copy(fx[fs], acc_sh.at[fbd[j].at[pos]], ss[fs],
                             add=True)

        def s_wait(fs, j, pos):
            pltpu.make_async_copy(fx[fs], acc_sh.at[fbd[j].at[pos]],
                                  ss[fs]).wait()

        def scale(rb, j, fs):
            rref = rbf[rb]
            oref = fx[fs]
            vref = fbv[j]

            @plsc.parallel_loop(0, SCH, unroll=8)
            def _scale(e):
                vv = plsc.load_gather(
                    vref, [jnp.full((16,), e + rb * SCH, jnp.int32)])
                for k in range(D // 32):
                    w = plsc.bitcast(rref[e, pl.ds(k * 32, 32)], jnp.int32)
                    ev = plsc.bitcast(w << 16, jnp.float32) * vv
                    od = plsc.bitcast(
                        w & jnp.int32(-65536), jnp.float32) * vv
                    plsc.store_scatter(oref.at[e], [idx_ev + k * 32], ev)
                    plsc.store_scatter(oref.at[e], [idx_od + k * 32], od)

        f_issue(0, 0)
        f_issue(1, 1)
        f_wait(0, 0)
        g_issue(0, 0)
        g_issue(1, 0)

        def block(p, _):
            for i in range(12):
                rb = i % 4
                j = i // 4
                rb2 = (i + 2) % 4
                j2 = ((i + 2) // 4) % 3
                fs = i % 2
                jw = 2 if i < 2 else (i - 2) // 4
                if i == 2:
                    f_wait(1, 3 * p + 1)
                if i == 6:
                    f_wait(2, 3 * p + 2)
                if i == 10:
                    @pl.when(p + 1 < nblk)
                    def _():
                        f_wait(0, 3 * p + 3)
                g_wait(rb, j)
                if i < 10:
                    g_issue(rb2, j2)
                else:
                    @pl.when(p + 1 < nblk)
                    def _():
                        g_issue(rb2, j2)
                if i < 2:
                    @pl.when(p > 0)
                    def _():
                        s_wait(fs, jw, rb2)
                else:
                    s_wait(fs, jw, rb2)
                scale(rb, j, fs)
                s_issue(fs, j, rb)
                if i == 2:
                    f_issue(2, 3 * p + 2)
                if i == 6:
                    @pl.when(p + 1 < nblk)
                    def _():
                        f_issue(0, 3 * p + 3)
                if i == 10:
                    @pl.when(p + 1 < nblk)
                    def _():
                        f_issue(1, 3 * p + 4)
            return 0

        lax.fori_loop(0, nblk, block, 0)
        s_wait(0, 2, 2)
        s_wait(1, 2, 3)

    plsc.subcore_barrier()

    pltpu.sync_copy(acc_sh.at[pl.ds(s * ROWS_PER_TILE, ROWS_PER_TILE)],
                    y_hbm.at[c, pl.ds(s * ROWS_PER_TILE, ROWS_PER_TILE)])


def _spmm_sc(x, srcp, dstp, valp, cnts):
    mesh = plsc.VectorSubcoreMesh(core_axis_name="c", subcore_axis_name="s")
    k = pl.kernel(
        _spmm_body,
        out_type=jax.ShapeDtypeStruct((NUM_SC, ACC_ROWS, D), jnp.float32),
        mesh=mesh,
        scratch_types=(
            [pltpu.VMEM((SCH, D), jnp.bfloat16)] * 4
            + [pltpu.VMEM((SCH, D), jnp.float32)] * 2
            + [pltpu.VMEM((SUP,), jnp.int32)] * 3
            + [pltpu.VMEM((SUP,), jnp.float32)] * 3
            + [pltpu.VMEM((4, SCH), jnp.int32)] * 3
            + [pltpu.VMEM((16,), jnp.int32)]
            + [pltpu.VMEM_SHARED((ACC_ROWS, D), jnp.float32)]
            + [pltpu.SemaphoreType.DMA] * 9
        ),
        compiler_params=_sc_compiler_params(),
    )
    return k(x, srcp, dstp, valp, cnts)


def _rescale_body(y_ref, ego_ref, acc_ref, x_ref, accout_ref):
    y = y_ref[0]
    e = ego_ref[...]
    num = jnp.sum(y * e, axis=1, keepdims=True)
    n1 = jnp.maximum(jnp.sqrt(jnp.sum(y * y, axis=1, keepdims=True)), 1e-8)
    n2 = jnp.maximum(jnp.sqrt(jnp.sum(e * e, axis=1, keepdims=True)), 1e-8)
    w = num / (n1 * n2)
    xn = w * y
    x_ref[...] = xn.astype(jnp.bfloat16)
    accout_ref[...] = acc_ref[...] + xn


_RBLK = 5000


def _rescale(y2, ego, acc):
    return pl.pallas_call(
        _rescale_body,
        grid=(2, HALF // _RBLK),
        in_specs=[
            pl.BlockSpec((1, _RBLK, D), lambda c, j: (c, j, 0)),
            pl.BlockSpec((_RBLK, D), lambda c, j: (c * (HALF // _RBLK) + j, 0)),
            pl.BlockSpec((_RBLK, D), lambda c, j: (c * (HALF // _RBLK) + j, 0)),
        ],
        out_specs=[
            pl.BlockSpec((_RBLK, D), lambda c, j: (c * (HALF // _RBLK) + j, 0)),
            pl.BlockSpec((_RBLK, D), lambda c, j: (c * (HALF // _RBLK) + j, 0)),
        ],
        out_shape=[
            jax.ShapeDtypeStruct((N_NODES, D), jnp.bfloat16),
            jax.ShapeDtypeStruct((N_NODES, D), jnp.float32),
        ],
    )(y2, ego, acc)


def kernel(features, id_embd, adj_indices, adj_values, W1, b1, W2, b2, preference):
    x0, x0b = _dense_prologue(features, id_embd, W1, b1, W2, b2, preference)
    dst = adj_indices[0].astype(jnp.int32)
    src = adj_indices[1].astype(jnp.int32)
    val = adj_values
    srcp, dstp, valp, cnts = _partition_sc(src, dst, val)
    xb = x0b
    acc = x0
    for _ in range(NUM_LAYER):
        y2 = _spmm_sc(xb, srcp, dstp, valp, cnts)
        xb, acc = _rescale(y2, x0, acc)
    return (acc, preference)


# speedup vs baseline: 1.5213x; 1.0010x over previous
import dataclasses

import jax
import jax.numpy as jnp
from jax import lax
from jax.experimental import pallas as pl
from jax.experimental.pallas import tpu as pltpu
from jax.experimental.pallas import tpu_sc as plsc

NUM_USER = 25000
NUM_ITEM = 25000
N_NODES = NUM_USER + NUM_ITEM
NUM_LAYER = 3
D = 64
E_EDGES = 800000

NUM_SC = 2
NUM_SUBCORES = 16
HALF = N_NODES // NUM_SC
ACC_ROWS = 25088
ROWS_PER_TILE = ACC_ROWS // NUM_SUBCORES
CHUNK = 400
NUM_TILES = NUM_SC * NUM_SUBCORES
N_QCHUNKS = E_EDGES // CHUNK
SCH = 96
SUP = 4 * SCH
BLK_E = 12 * SCH
FL = BLK_E
CAP = 23 * FL
STG = 2 * FL

_BLK = 1000
_GRID = NUM_ITEM // _BLK


def _row_normalize(v):
    n = jnp.sqrt(jnp.sum(v * v, axis=1, keepdims=True))
    return v / jnp.maximum(n, 1e-12)


def _prologue_body(f_ref, id_ref, w1_ref, b1_ref, w2_ref, b2_ref, p_ref, o_ref, ob_ref):
    f = f_ref[...]
    t = jnp.dot(f, w1_ref[...], preferred_element_type=jnp.float32) + b1_ref[...]
    t = jnp.where(t >= 0, t, 0.01 * t)
    t = jnp.dot(t, w2_ref[...], preferred_element_type=jnp.float32) + b2_ref[...]
    idb = id_ref[...]
    t = jnp.sqrt(jnp.abs((idb * idb + t * t) * 0.5 + 1e-8))
    xp = _row_normalize(p_ref[...])
    xt = _row_normalize(t)
    o_ref[0] = xp
    o_ref[1] = xt
    ob_ref[0] = xp.astype(jnp.bfloat16)
    ob_ref[1] = xt.astype(jnp.bfloat16)


def _dense_prologue(features, id_embd, W1, b1, W2, b2, preference):
    x2 = pl.pallas_call(
        _prologue_body,
        grid=(_GRID,),
        in_specs=[
            pl.BlockSpec((_BLK, 128), lambda i: (i, 0)),
            pl.BlockSpec((_BLK, D), lambda i: (i, 0)),
            pl.BlockSpec((128, 256), lambda i: (0, 0)),
            pl.BlockSpec((1, 256), lambda i: (0, 0)),
            pl.BlockSpec((256, D), lambda i: (0, 0)),
            pl.BlockSpec((1, D), lambda i: (0, 0)),
            pl.BlockSpec((_BLK, D), lambda i: (i, 0)),
        ],
        out_specs=[pl.BlockSpec((2, _BLK, D), lambda i: (0, i, 0)),
                   pl.BlockSpec((2, _BLK, D), lambda i: (0, i, 0))],
        out_shape=[jax.ShapeDtypeStruct((2, NUM_ITEM, D), jnp.float32),
                   jax.ShapeDtypeStruct((2, NUM_ITEM, D), jnp.bfloat16)],
    )(features, id_embd, W1, b1.reshape(1, -1), W2, b2.reshape(1, -1), preference)
    x2, xb2 = x2
    return x2.reshape(N_NODES, D), xb2.reshape(N_NODES, D)


def _sc_compiler_params():
    cp = pltpu.CompilerParams()
    if "needs_layout_passes" in pltpu.CompilerParams.__dataclass_fields__:
        cp = dataclasses.replace(cp, needs_layout_passes=False)
    if "use_tc_tiling_on_sc" in pltpu.CompilerParams.__dataclass_fields__:
        cp = dataclasses.replace(cp, use_tc_tiling_on_sc=False)
    return cp


def _partition_body(src_hbm, dst_hbm, val_hbm,
                    srcp_hbm, dstp_hbm, valp_hbm, cnts_hbm,
                    src_v, dst_v, val_v,
                    stg_s0, stg_d0, stg_v0, stg_s1, stg_d1, stg_v1,
                    cnt_v, sem_s, sem_d, sem_v):
    c = lax.axis_index("c")
    s = lax.axis_index("s")
    w = c * NUM_SUBCORES + s
    nch = jnp.where(w < N_QCHUNKS - 62 * NUM_TILES, 63, 62)

    stgs = (stg_s0, stg_s1)
    stgd = (stg_d0, stg_d1)
    stgv = (stg_v0, stg_v1)

    def flush(h, off):
        off = pl.multiple_of(off, FL)
        pltpu.sync_copy(stgs[h].at[pl.ds(0, FL)],
                        srcp_hbm.at[h, w, pl.ds(off, FL)])
        pltpu.sync_copy(stgd[h].at[pl.ds(0, FL)],
                        dstp_hbm.at[h, w, pl.ds(off, FL)])
        pltpu.sync_copy(stgv[h].at[pl.ds(0, FL)],
                        valp_hbm.at[h, w, pl.ds(off, FL)])

    def shift(h):
        @pl.loop(0, FL // 16)
        def _sh(i):
            sl_from = pl.ds(FL + i * 16, 16)
            sl_to = pl.ds(i * 16, 16)
            stgs[h][sl_to] = stgs[h][sl_from]
            stgd[h][sl_to] = stgd[h][sl_from]
            stgv[h][sl_to] = stgv[h][sl_from]

    def chunk_body(j, carry):
        c0, c1, o0, o1 = carry
        off_in = (w + j * NUM_TILES) * CHUNK
        cp_s = pltpu.async_copy(src_hbm.at[pl.ds(off_in, CHUNK)], src_v, sem_s)
        cp_d = pltpu.async_copy(dst_hbm.at[pl.ds(off_in, CHUNK)], dst_v, sem_d)
        cp_v = pltpu.async_copy(val_hbm.at[pl.ds(off_in, CHUNK)], val_v, sem_v)
        cp_s.wait()
        cp_d.wait()
        cp_v.wait()

        def slice_body(i, cc):
            c0i, c1i = cc
            sl = pl.ds(i * 16, 16)
            dd = dst_v[sl]
            ss = src_v[sl]
            vv = val_v[sl]
            m0 = dd < HALF
            m1 = jnp.logical_not(m0)
            plsc.store_compressed(stg_s0.at[pl.ds(c0i, 16)], ss, mask=m0)
            plsc.store_compressed(stg_d0.at[pl.ds(c0i, 16)], dd, mask=m0)
            plsc.store_compressed(stg_v0.at[pl.ds(c0i, 16)], vv, mask=m0)
            plsc.store_compressed(stg_s1.at[pl.ds(c1i, 16)], ss, mask=m1)
            plsc.store_compressed(stg_d1.at[pl.ds(c1i, 16)], dd - HALF, mask=m1)
            plsc.store_compressed(stg_v1.at[pl.ds(c1i, 16)], vv, mask=m1)
            n0 = jnp.sum(m0.astype(jnp.int32))
            return (c0i + n0, c1i + (16 - n0))

        c0, c1 = lax.fori_loop(0, CHUNK // 16, slice_body, (c0, c1))

        f0 = c0 >= FL

        @pl.when(f0)
        def _():
            flush(0, o0)
            shift(0)

        o0 = o0 + jnp.where(f0, FL, 0)
        c0 = c0 - jnp.where(f0, FL, 0)

        f1 = c1 >= FL

        @pl.when(f1)
        def _():
            flush(1, o1)
            shift(1)

        o1 = o1 + jnp.where(f1, FL, 0)
        c1 = c1 - jnp.where(f1, FL, 0)
        return (c0, c1, o0, o1)

    c0, c1, o0, o1 = lax.fori_loop(
        0, nch, chunk_body, (jnp.int32(0), jnp.int32(0), jnp.int32(0), jnp.int32(0)))

    for h in range(2):
        cc = (c0, c1)[h]
        oo = (o0, o1)[h]

        @pl.loop(0, FL // 16)
        def _pad(i, _h=h, _cc=cc):
            sl = pl.ds(_cc + i * 16, 16)
            stgs[_h][sl] = jnp.zeros((16,), jnp.int32)
            stgd[_h][sl] = jnp.full((16,), HALF, jnp.int32)
            stgv[_h][sl] = jnp.zeros((16,), jnp.float32)

        flush(h, oo)
        cnt_v[pl.ds(0, 16)] = jnp.full((16,), oo + FL, jnp.int32)
        pltpu.sync_copy(cnt_v.at[pl.ds(0, 8)], cnts_hbm.at[h, w])


def _partition_sc(src, dst, val):
    mesh = plsc.VectorSubcoreMesh(core_axis_name="c", subcore_axis_name="s")
    k = pl.kernel(
        _partition_body,
        out_type=[
            jax.ShapeDtypeStruct((2, NUM_TILES, CAP), jnp.int32),
            jax.ShapeDtypeStruct((2, NUM_TILES, CAP), jnp.int32),
            jax.ShapeDtypeStruct((2, NUM_TILES, CAP), jnp.float32),
            jax.ShapeDtypeStruct((2, NUM_TILES, 8), jnp.int32),
        ],
        mesh=mesh,
        scratch_types=[
            pltpu.VMEM((CHUNK,), jnp.int32),
            pltpu.VMEM((CHUNK,), jnp.int32),
            pltpu.VMEM((CHUNK,), jnp.float32),
            pltpu.VMEM((STG,), jnp.int32),
            pltpu.VMEM((STG,), jnp.int32),
            pltpu.VMEM((STG,), jnp.float32),
            pltpu.VMEM((STG,), jnp.int32),
            pltpu.VMEM((STG,), jnp.int32),
            pltpu.VMEM((STG,), jnp.float32),
            pltpu.VMEM((16,), jnp.int32),
            pltpu.SemaphoreType.DMA,
            pltpu.SemaphoreType.DMA,
            pltpu.SemaphoreType.DMA,
        ],
        compiler_params=_sc_compiler_params(),
    )
    return k(src, dst, val)


def _spmm_body(x_hbm, srcp_hbm, dstp_hbm, valp_hbm, cnts_hbm, y_hbm,
               rb_0, rb_1, rb_2, rb_3, fx_0, fx_1,
               fbs_0, fbs_1, fbs_2, fbv_0, fbv_1, fbv_2,
               fbd_0, fbd_1, fbd_2, cnt_v, acc_sh,
               sg_0, sg_1, sg_2, sg_3, ss_0, ss_1,
               sf_0, sf_1, sf_2):
    c = lax.axis_index("c")
    s = lax.axis_index("s")
    rbf = (rb_0, rb_1, rb_2, rb_3)
    fx = (fx_0, fx_1)
    fbs = (fbs_0, fbs_1, fbs_2)
    fbv = (fbv_0, fbv_1, fbv_2)
    fbd = (fbd_0, fbd_1, fbd_2)
    sg = (sg_0, sg_1, sg_2, sg_3)
    ss = (ss_0, ss_1)
    sf = (sf_0, sf_1, sf_2)

    iota16 = lax.broadcasted_iota(jnp.int32, (16,), 0)
    idx_ev = iota16 * 2
    idx_od = idx_ev + 1

    @pl.loop(0, SCH)
    def _zero_rows(i):
        for k in range(D // 16):
            fx_0[i, pl.ds(k * 16, 16)] = jnp.zeros((16,), jnp.float32)

    for r in range(ROWS_PER_TILE // SCH):
        pltpu.sync_copy(fx_0, acc_sh.at[pl.ds(s * ROWS_PER_TILE + r * SCH, SCH)])
    _REM = ROWS_PER_TILE - (ROWS_PER_TILE // SCH) * SCH
    if _REM:
        pltpu.sync_copy(fx_0.at[pl.ds(0, _REM)],
                        acc_sh.at[pl.ds(s * ROWS_PER_TILE
                                        + (ROWS_PER_TILE // SCH) * SCH, _REM)])
    plsc.subcore_barrier()

    for rr in range(2):
        region = s * 2 + rr
        pltpu.sync_copy(cnts_hbm.at[c, region], cnt_v.at[pl.ds(0, 8)])
        cnt = cnt_v[pl.ds(0, 16)][0]
        nblk = cnt // BLK_E

        def f_issue(j, sup):
            pltpu.async_copy(srcp_hbm.at[c, region, pl.ds(sup * SUP, SUP)],
                             fbs[j], sf[j])
            pltpu.async_copy(valp_hbm.at[c, region, pl.ds(sup * SUP, SUP)],
                             fbv[j], sf[j])
            for q in range(4):
                pltpu.async_copy(
                    dstp_hbm.at[c, region, pl.ds(sup * SUP + q * SCH, SCH)],
                    fbd[j].at[q], sf[j])

        def f_wait(j, sup):
            pltpu.make_async_copy(srcp_hbm.at[c, region, pl.ds(sup * SUP, SUP)],
                                  fbs[j], sf[j]).wait()
            pltpu.make_async_copy(valp_hbm.at[c, region, pl.ds(sup * SUP, SUP)],
                                  fbv[j], sf[j]).wait()
            for q in range(4):
                pltpu.make_async_copy(
                    dstp_hbm.at[c, region, pl.ds(sup * SUP + q * SCH, SCH)],
                    fbd[j].at[q], sf[j]).wait()

        def g_issue(rb, j):
            pltpu.async_copy(x_hbm.at[fbs[j].at[pl.ds(rb * SCH, SCH)]],
                             rbf[rb], sg[rb])

        def g_wait(rb, j):
            pltpu.make_async_copy(x_hbm.at[fbs[j].at[pl.ds(rb * SCH, SCH)]],
                                  rbf[rb], sg[rb]).wait()

        def s_issue(fs, j, pos):
            pltpu.async_copy(fx[fs], acc_sh.at[fbd[j].at[pos]], ss[fs],
                             add=True)

        def s_wait(fs, j, pos):
            pltpu.make_async_copy(fx[fs], acc_sh.at[fbd[j].at[pos]],
                                  ss[fs]).wait()

        def scale(rb, j, fs):
            rref = rbf[rb]
            oref = fx[fs]
            vref = fbv[j]

            @plsc.parallel_loop(0, SCH, unroll=8)
            def _scale(e):
                vv = plsc.load_gather(
                    vref, [jnp.full((16,), e + rb * SCH, jnp.int32)])
                for k in range(D // 32):
                    w = plsc.bitcast(rref[e, pl.ds(k * 32, 32)], jnp.int32)
                    ev = plsc.bitcast(w << 16, jnp.float32) * vv
                    od = plsc.bitcast(
                        w & jnp.int32(-65536), jnp.float32) * vv
                    plsc.store_scatter(oref.at[e], [idx_ev + k * 32], ev)
                    plsc.store_scatter(oref.at[e], [idx_od + k * 32], od)

        f_issue(0, 0)
        f_issue(1, 1)
        f_wait(0, 0)
        g_issue(0, 0)
        g_issue(1, 0)

        def block(p, _):
            for i in range(12):
                rb = i % 4
                j = i // 4
                rb2 = (i + 2) % 4
                j2 = ((i + 2) // 4) % 3
                fs = i % 2
                jw = 2 if i < 2 else (i - 2) // 4
                if i == 2:
                    f_wait(1, 3 * p + 1)
                if i == 6:
                    f_wait(2, 3 * p + 2)
                if i == 10:
                    @pl.when(p + 1 < nblk)
                    def _():
                        f_wait(0, 3 * p + 3)
                g_wait(rb, j)
                if i < 10:
                    g_issue(rb2, j2)
                else:
                    @pl.when(p + 1 < nblk)
                    def _():
                        g_issue(rb2, j2)
                if i < 2:
                    @pl.when(p > 0)
                    def _():
                        s_wait(fs, jw, rb2)
                else:
                    s_wait(fs, jw, rb2)
                scale(rb, j, fs)
                s_issue(fs, j, rb)
                if i == 2:
                    f_issue(2, 3 * p + 2)
                if i == 6:
                    @pl.when(p + 1 < nblk)
                    def _():
                        f_issue(0, 3 * p + 3)
                if i == 10:
                    @pl.when(p + 1 < nblk)
                    def _():
                        f_issue(1, 3 * p + 4)
            return 0

        lax.fori_loop(0, nblk, block, 0)
        s_wait(0, 2, 2)
        s_wait(1, 2, 3)

    plsc.subcore_barrier()

    pltpu.sync_copy(acc_sh.at[pl.ds(s * ROWS_PER_TILE, ROWS_PER_TILE)],
                    y_hbm.at[c, pl.ds(s * ROWS_PER_TILE, ROWS_PER_TILE)])


def _spmm_sc(x, srcp, dstp, valp, cnts):
    mesh = plsc.VectorSubcoreMesh(core_axis_name="c", subcore_axis_name="s")
    k = pl.kernel(
        _spmm_body,
        out_type=jax.ShapeDtypeStruct((NUM_SC, ACC_ROWS, D), jnp.float32),
        mesh=mesh,
        scratch_types=(
            [pltpu.VMEM((SCH, D), jnp.bfloat16)] * 4
            + [pltpu.VMEM((SCH, D), jnp.float32)] * 2
            + [pltpu.VMEM((SUP,), jnp.int32)] * 3
            + [pltpu.VMEM((SUP,), jnp.float32)] * 3
            + [pltpu.VMEM((4, SCH), jnp.int32)] * 3
            + [pltpu.VMEM((16,), jnp.int32)]
            + [pltpu.VMEM_SHARED((ACC_ROWS, D), jnp.float32)]
            + [pltpu.SemaphoreType.DMA] * 9
        ),
        compiler_params=_sc_compiler_params(),
    )
    return k(x, srcp, dstp, valp, cnts)


def _rescale_body(y_ref, ego_ref, acc_ref, x_ref, accout_ref):
    y = y_ref[0]
    e = ego_ref[...]
    num = jnp.sum(y * e, axis=1, keepdims=True)
    n1 = jnp.maximum(jnp.sqrt(jnp.sum(y * y, axis=1, keepdims=True)), 1e-8)
    n2 = jnp.maximum(jnp.sqrt(jnp.sum(e * e, axis=1, keepdims=True)), 1e-8)
    w = num / (n1 * n2)
    xn = w * y
    x_ref[...] = xn.astype(jnp.bfloat16)
    accout_ref[...] = acc_ref[...] + xn


_RBLK = 5000


def _rescale(y2, ego, acc):
    return pl.pallas_call(
        _rescale_body,
        grid=(2, HALF // _RBLK),
        in_specs=[
            pl.BlockSpec((1, _RBLK, D), lambda c, j: (c, j, 0)),
            pl.BlockSpec((_RBLK, D), lambda c, j: (c * (HALF // _RBLK) + j, 0)),
            pl.BlockSpec((_RBLK, D), lambda c, j: (c * (HALF // _RBLK) + j, 0)),
        ],
        out_specs=[
            pl.BlockSpec((_RBLK, D), lambda c, j: (c * (HALF // _RBLK) + j, 0)),
            pl.BlockSpec((_RBLK, D), lambda c, j: (c * (HALF // _RBLK) + j, 0)),
        ],
        out_shape=[
            jax.ShapeDtypeStruct((N_NODES, D), jnp.bfloat16),
            jax.ShapeDtypeStruct((N_NODES, D), jnp.float32),
        ],
    )(y2, ego, acc)


def kernel(features, id_embd, adj_indices, adj_values, W1, b1, W2, b2, preference):
    x0, x0b = _dense_prologue(features, id_embd, W1, b1, W2, b2, preference)
    dst = adj_indices[0].astype(jnp.int32)
    src = adj_indices[1].astype(jnp.int32)
    val = adj_values
    srcp, dstp, valp, cnts = _partition_sc(src, dst, val)
    xb = x0b
    acc = x0
    for _ in range(NUM_LAYER):
        y2 = _spmm_sc(xb, srcp, dstp, valp, cnts)
        xb, acc = _rescale(y2, x0, acc)
    return (acc, preference)
